# Initial kernel scaffold; baseline (speedup 1.0000x reference)
import jax
import jax.numpy as jnp
from jax.experimental import pallas as pl


def kernel(data, data_feature, Ws, bs, gammas, betas):
    raise NotImplementedError("write your pallas kernel here")


# speedup vs baseline: 1.5496x; 1.5496x over previous
import functools

import jax
import jax.numpy as jnp
from jax.experimental import pallas as pl
from jax.experimental.pallas import tpu as pltpu

_K = 32
_S = 1024
_EPS = 1e-5


def _fps_kernel(data_ref, idx_ref, cen_ref, dist_ref):
    B, N = dist_ref.shape
    S = idx_ref.shape[1]
    X = data_ref[0]
    Y = data_ref[1]
    Z = data_ref[2]
    iota_n = jax.lax.broadcasted_iota(jnp.int32, (B, N), 1)
    iota_s = jax.lax.broadcasted_iota(jnp.int32, (B, S), 1)
    dist_ref[...] = jnp.full((B, N), 1e10, jnp.float32)
    idx_ref[...] = jnp.zeros((B, S), jnp.int32)
    cen_ref[...] = jnp.zeros((3, B, S), jnp.float32)

    def body(i, far):
        onehot = iota_n == far
        cx = jnp.max(jnp.where(onehot, X, -jnp.inf), axis=1, keepdims=True)
        cy = jnp.max(jnp.where(onehot, Y, -jnp.inf), axis=1, keepdims=True)
        cz = jnp.max(jnp.where(onehot, Z, -jnp.inf), axis=1, keepdims=True)
        sel = iota_s == i
        idx_ref[...] = jnp.where(sel, far, idx_ref[...])
        cen_ref[0] = jnp.where(sel, cx, cen_ref[0])
        cen_ref[1] = jnp.where(sel, cy, cen_ref[1])
        cen_ref[2] = jnp.where(sel, cz, cen_ref[2])
        dx = X - cx
        dy = Y - cy
        dz = Z - cz
        d = dx * dx + dy * dy + dz * dz
        dist = jnp.minimum(dist_ref[...], d)
        dist_ref[...] = dist
        maxv = jnp.max(dist, axis=1, keepdims=True)
        far_next = jnp.min(jnp.where(dist == maxv, iota_n, N), axis=1,
                           keepdims=True)
        return far_next

    jax.lax.fori_loop(0, S, body, jnp.zeros((B, 1), jnp.int32))


def _fps_pallas(data):
    B, N, C = data.shape
    data_t = jnp.transpose(data, (2, 0, 1))
    idx, cen = pl.pallas_call(
        _fps_kernel,
        out_shape=[
            jax.ShapeDtypeStruct((B, _S), jnp.int32),
            jax.ShapeDtypeStruct((3, B, _S), jnp.float32),
        ],
        scratch_shapes=[pltpu.VMEM((B, N), jnp.float32)],
    )(data_t)
    return idx, jnp.transpose(cen, (1, 2, 0))


def _index_pts(data, idx):
    if idx.ndim == 2:
        return jnp.take_along_axis(data, idx[:, :, None], axis=1)
    B, S, K = idx.shape
    flat = idx.reshape(B, S * K)
    out = jnp.take_along_axis(data, flat[:, :, None], axis=1)
    return out.reshape(B, S, K, data.shape[-1])


def _square_dist(a, b):
    B, N, _ = a.shape
    _, M, _ = b.shape
    ans = -2.0 * jnp.matmul(a, jnp.transpose(b, (0, 2, 1)))
    ans = ans + jnp.sum(a ** 2, -1).reshape(B, N, 1)
    ans = ans + jnp.sum(b ** 2, -1).reshape(B, 1, M)
    return ans


def kernel(data, data_feature, Ws, bs, gammas, betas):
    B, N, C = data.shape
    fps_idx, centroids = _fps_pallas(data)
    dist = _square_dist(centroids, data)
    _, idx2k = jax.lax.top_k(dist, 2 * _K)
    sample = _index_pts(data, idx2k)
    diff = sample - jnp.roll(sample, 1, axis=2)
    score = jnp.abs(diff)[:, :, :, 2]
    _, topk_idx = jax.lax.top_k(score, _K)
    sample = jnp.take_along_axis(sample, topk_idx[..., None], axis=2)
    sample_norm = sample - centroids[:, :, None, :]
    tmp = _index_pts(data_feature, topk_idx)
    ans = jnp.concatenate([sample_norm, tmp], axis=-1)
    z = jnp.abs(sample[..., -1] - centroids[:, :, None, -1])
    z = jax.nn.softmax(z, axis=-1)
    ans = ans * z[..., None]
    x = jnp.transpose(ans, (0, 3, 2, 1))
    for W, b, g, be in zip(Ws, bs, gammas, betas):
        x = jnp.einsum('oc,bcks->boks', W, x) + b[None, :, None, None]
        mean = jnp.mean(x, axis=(0, 2, 3), keepdims=True)
        var = jnp.var(x, axis=(0, 2, 3), keepdims=True)
        x = (x - mean) / jnp.sqrt(var + _EPS) * g[None, :, None, None] + be[None, :, None, None]
        x = jax.nn.relu(x)
    x = jnp.max(x, axis=2)
    return centroids, jnp.transpose(x, (0, 2, 1))
